# baseline (speedup 1.0000x reference)
import functools

import jax
import jax.numpy as jnp
from jax.experimental import pallas as pl
from jax.experimental.pallas import tpu as pltpu
from jax.experimental.pallas import tpu_sc as plsc

_BLK = 2000
_GW = 1600


def _f32dot(a, b):
    return jnp.dot(a, b, preferred_element_type=jnp.float32)


def _ln(x, s, b):
    mu = jnp.mean(x, axis=-1, keepdims=True)
    xc = x - mu
    var = jnp.mean(xc * xc, axis=-1, keepdims=True)
    return xc * jax.lax.rsqrt(var + 1e-6) * s + b


def _silu(x):
    return x * (0.5 + 0.5 * jnp.tanh(0.5 * x))


def _softplus(x):
    return jnp.maximum(x, 0.0) + jnp.log1p(jnp.exp(-jnp.abs(x)))


def _embed_body(atom_ref, We_ref, be_ref, lns_ref, lnb_ref, W1s_ref, W1n_ref,
                ba_ref, x_ref, a_ref, g_ref):
    x = _f32dot(atom_ref[...], We_ref[...]) + be_ref[...]
    x_ref[...] = x
    xn = _ln(x, lns_ref[...], lnb_ref[...])
    a_ref[...] = _f32dot(xn, W1s_ref[...]) + ba_ref[...]
    g_ref[...] = _f32dot(xn, W1n_ref[...])


def _edge_agg(nbr_ref, gath_ref, a_ref, KC_ref, A8_ref, W2S_ref):
    blk4 = nbr_ref.shape[0]
    blk = blk4 // 4
    aterm = _f32dot(a_ref[...].astype(jnp.bfloat16), A8_ref[...])
    aterm4 = jnp.broadcast_to(aterm[:, None, :], (blk, 4, 128))
    pre = (_f32dot(nbr_ref[...], KC_ref[...]) + gath_ref[...]
           + aterm4.reshape(blk4, 128))
    s = _silu(pre)
    ssum = jnp.sum(s.reshape(blk, 4, 128), axis=1)
    return _f32dot(ssum.astype(jnp.bfloat16), W2S_ref[...])


def _layer_body(nbr_ref, gath_ref, a_ref, x_ref, KC_ref, A8_ref, W2S_ref,
                bx_ref, lns_ref, lnb_ref, W1s_ref, W1n_ref, ba_ref,
                x1_ref, a1_ref, g1_ref):
    agg = _edge_agg(nbr_ref, gath_ref, a_ref, KC_ref, A8_ref, W2S_ref)
    x1 = x_ref[...] + agg + bx_ref[...]
    x1_ref[...] = x1
    xn = _ln(x1, lns_ref[...], lnb_ref[...])
    a1_ref[...] = _f32dot(xn, W1s_ref[...]) + ba_ref[...]
    g1_ref[...] = _f32dot(xn, W1n_ref[...])


def _final_body(nbr_ref, gath_ref, a_ref, x_ref, KC_ref, A8_ref, W2S_ref,
                bx_ref, Wr1_ref, br1_ref, Wr2_ref, br2_ref, Wr3_ref, br3_ref,
                acc_ref):
    agg = _edge_agg(nbr_ref, gath_ref, a_ref, KC_ref, A8_ref, W2S_ref)
    x2 = x_ref[...] + agg + bx_ref[...]
    h = _softplus(_f32dot(x2, Wr1_ref[...]) + br1_ref[...])
    t = _softplus(_f32dot(h, Wr2_ref[...]) + br2_ref[...])
    part = (jnp.sum(_f32dot(t, Wr3_ref[...]), keepdims=True)
            + t.shape[0] * br3_ref[...])
    i = pl.program_id(0)

    @pl.when(i == 0)
    def _():
        acc_ref[...] = part

    @pl.when(i > 0)
    def _():
        acc_ref[...] += part


def _sc_gather(table, idx_flat):
    num = idx_flat.shape[0]
    af = table.shape[1]
    mesh = plsc.VectorSubcoreMesh(core_axis_name="c", subcore_axis_name="s")
    idx2 = idx_flat.reshape(num // _GW, _GW)

    @functools.partial(
        pl.kernel,
        out_type=jax.ShapeDtypeStruct((num, af), table.dtype),
        mesh=mesh,
        compiler_params=pltpu.CompilerParams(use_tc_tiling_on_sc=False),
    )
    def k(table_hbm, i_hbm, o_hbm):
        def body(i_vmem, o_vmem):
            pltpu.sync_copy(table_hbm.at[i_vmem.at[0]], o_vmem)

        pltpu.emit_pipeline(
            body,
            grid=(num // _GW,),
            in_specs=[pl.BlockSpec((1, _GW), lambda i: (i, 0))],
            out_specs=[pl.BlockSpec((_GW, af), lambda i: (i, 0))],
            core_axis_name=("c", "s"),
            dimension_semantics=(pltpu.PARALLEL,),
        )(i_hbm, o_hbm)

    return k(table, idx2)


def kernel(atom_fea, nbr_fea, nbr_fea_idx,
           W_embed, b_embed, W_edge, b_edge,
           ln0_s, ln0_b, W1_0, b1_0, W2_0, b2_0,
           ln1_s, ln1_b, W1_1, b1_1, W2_1, b2_1,
           Wr1, br1, Wr2, br2, Wr3, br3):
    n, d_in = atom_fea.shape
    m = nbr_fea.shape[1]
    af = W_embed.shape[1]
    maf = m * af
    nsub = maf // 128
    n4 = n * nsub
    nblk = n // _BLK
    blk4 = _BLK * nsub

    nbr4 = nbr_fea.reshape(n4, 128).astype(jnp.bfloat16)
    idx_flat = nbr_fea_idx.reshape(-1)

    eye = jnp.eye(af, dtype=jnp.float32)
    A8 = jnp.tile(eye, (1, 128 // af)).astype(jnp.bfloat16)

    def layer_consts(W1, b1, W2, b2):
        W1s, W1n, W1e = W1[:af], W1[af:2 * af], W1[2 * af:]
        C = W_edge @ W1e
        KC = jnp.kron(jnp.eye(128 // af, dtype=jnp.float32),
                      C).astype(jnp.bfloat16)
        W2S = jnp.tile(W2, (128 // af, 1)).astype(jnp.bfloat16)
        ba = (b1 + b_edge @ W1e).reshape(1, af)
        bx = (m * b2).reshape(1, af)
        return W1s, W1n, KC, W2S, ba, bx

    W1s0, W1n0, KC0, W2S0, ba0, bx0 = layer_consts(W1_0, b1_0, W2_0, b2_0)
    W1s1, W1n1, KC1, W2S1, ba1, bx1 = layer_consts(W1_1, b1_1, W2_1, b2_1)

    row = lambda shp: pl.BlockSpec(shp, lambda i: (i, 0))
    full = lambda shp: pl.BlockSpec(shp, lambda i: (0, 0))
    b16 = [jax.ShapeDtypeStruct((n, af), jnp.float32)] * 3

    x0, a0, g0 = pl.pallas_call(
        _embed_body,
        grid=(nblk,),
        in_specs=[row((_BLK, d_in)), full((d_in, af)), full((1, af)),
                  full((1, af)), full((1, af)), full((af, af)),
                  full((af, af)), full((1, af))],
        out_specs=[row((_BLK, af))] * 3,
        out_shape=b16,
    )(atom_fea, W_embed, b_embed.reshape(1, af),
      ln0_s.reshape(1, af), ln0_b.reshape(1, af), W1s0, W1n0, ba0)

    gath0 = _sc_gather(g0, idx_flat).reshape(n4, 128)

    x1, a1, g1 = pl.pallas_call(
        _layer_body,
        grid=(nblk,),
        in_specs=[row((blk4, 128)), row((blk4, 128)), row((_BLK, af)),
                  row((_BLK, af)), full((128, 128)), full((af, 128)),
                  full((128, af)), full((1, af)), full((1, af)),
                  full((1, af)), full((af, af)), full((af, af)),
                  full((1, af))],
        out_specs=[row((_BLK, af))] * 3,
        out_shape=b16,
    )(nbr4, gath0, a0, x0, KC0, A8, W2S0, bx0,
      ln1_s.reshape(1, af), ln1_b.reshape(1, af), W1s1, W1n1, ba1)

    gath1 = _sc_gather(g1, idx_flat).reshape(n4, 128)

    h = Wr1.shape[1]
    acc = pl.pallas_call(
        _final_body,
        grid=(nblk,),
        in_specs=[row((blk4, 128)), row((blk4, 128)), row((_BLK, af)),
                  row((_BLK, af)), full((128, 128)), full((af, 128)),
                  full((128, af)), full((1, af)),
                  full((af, h)), full((1, h)), full((h, h // 2)),
                  full((1, h // 2)), full((h // 2, 1)), full((1, 1))],
        out_specs=pl.BlockSpec((1, 1), lambda i: (0, 0)),
        out_shape=jax.ShapeDtypeStruct((1, 1), jnp.float32),
    )(nbr4, gath1, a1, x1, KC1, A8, W2S1, bx1,
      Wr1, br1.reshape(1, h), Wr2, br2.reshape(1, h // 2),
      Wr3, br3.reshape(1, 1))

    return acc[0, 0] / n


# speedup vs baseline: 1.0003x; 1.0003x over previous
import functools

import jax
import jax.numpy as jnp
from jax.experimental import pallas as pl
from jax.experimental.pallas import tpu as pltpu
from jax.experimental.pallas import tpu_sc as plsc

_BLK = 2000
_GW = 1600


def _f32dot(a, b):
    return jnp.dot(a, b, preferred_element_type=jnp.float32)


def _ln(x, s, b):
    mu = jnp.mean(x, axis=-1, keepdims=True)
    xc = x - mu
    var = jnp.mean(xc * xc, axis=-1, keepdims=True)
    return xc * jax.lax.rsqrt(var + 1e-6) * s + b


def _silu(x):
    return x * (0.5 + 0.5 * jnp.tanh(0.5 * x))


def _softplus(x):
    return jnp.maximum(x, 0.0) + jnp.log1p(jnp.exp(-jnp.abs(x)))


def _embed_body(atom_ref, We_ref, be_ref, lns_ref, lnb_ref, W1s_ref, W1n_ref,
                ba_ref, x_ref, a_ref, g_ref):
    x = _f32dot(atom_ref[...], We_ref[...]) + be_ref[...]
    x_ref[...] = x
    xn = _ln(x, lns_ref[...], lnb_ref[...])
    a_ref[...] = _f32dot(xn, W1s_ref[...]) + ba_ref[...]
    g_ref[...] = _f32dot(xn, W1n_ref[...])


def _edge_agg(nbr_ref, gath_ref, a_ref, KC_ref, A8_ref, W2S_ref):
    blk4 = nbr_ref.shape[0]
    blk = blk4 // 4
    aterm = _f32dot(a_ref[...].astype(jnp.bfloat16), A8_ref[...])
    aterm4 = jnp.broadcast_to(aterm[:, None, :], (blk, 4, 128))
    pre = (_f32dot(nbr_ref[...], KC_ref[...]) + gath_ref[...]
           + aterm4.reshape(blk4, 128))
    s = _silu(pre)
    ssum = jnp.sum(s.reshape(blk, 4, 128), axis=1)
    return _f32dot(ssum.astype(jnp.bfloat16), W2S_ref[...])


def _layer_body(nbr_ref, gath_ref, a_ref, x_ref, KC_ref, A8_ref, W2S_ref,
                bx_ref, lns_ref, lnb_ref, W1s_ref, W1n_ref, ba_ref,
                x1_ref, a1_ref, g1_ref):
    agg = _edge_agg(nbr_ref, gath_ref, a_ref, KC_ref, A8_ref, W2S_ref)
    x1 = x_ref[...] + agg + bx_ref[...]
    x1_ref[...] = x1
    xn = _ln(x1, lns_ref[...], lnb_ref[...])
    a1_ref[...] = _f32dot(xn, W1s_ref[...]) + ba_ref[...]
    g1_ref[...] = _f32dot(xn, W1n_ref[...])


def _final_body(nbr_ref, gath_ref, a_ref, x_ref, KC_ref, A8_ref, W2S_ref,
                bx_ref, Wr1_ref, br1_ref, Wr2_ref, br2_ref, Wr3_ref, br3_ref,
                acc_ref):
    agg = _edge_agg(nbr_ref, gath_ref, a_ref, KC_ref, A8_ref, W2S_ref)
    x2 = x_ref[...] + agg + bx_ref[...]
    h = _softplus(_f32dot(x2, Wr1_ref[...]) + br1_ref[...])
    t = _softplus(_f32dot(h, Wr2_ref[...]) + br2_ref[...])
    part = (jnp.sum(_f32dot(t, Wr3_ref[...]), keepdims=True)
            + t.shape[0] * br3_ref[...])
    i = pl.program_id(0)

    @pl.when(i == 0)
    def _():
        acc_ref[...] = part

    @pl.when(i > 0)
    def _():
        acc_ref[...] += part


def _sc_gather(table, idx_flat):
    num = idx_flat.shape[0]
    af = table.shape[1]
    mesh = plsc.VectorSubcoreMesh(core_axis_name="c", subcore_axis_name="s")
    idx2 = idx_flat.reshape(num // _GW, _GW)

    @functools.partial(
        pl.kernel,
        out_type=jax.ShapeDtypeStruct((num, af), table.dtype),
        mesh=mesh,
        compiler_params=pltpu.CompilerParams(use_tc_tiling_on_sc=False),
    )
    def k(table_hbm, i_hbm, o_hbm):
        def body(i_vmem, o_vmem):
            pltpu.sync_copy(table_hbm.at[i_vmem.at[0]], o_vmem)

        pltpu.emit_pipeline(
            body,
            grid=(num // _GW,),
            in_specs=[pl.BlockSpec((1, _GW), lambda i: (i, 0))],
            out_specs=[pl.BlockSpec((_GW, af), lambda i: (i, 0))],
            core_axis_name=("c", "s"),
            dimension_semantics=(pltpu.PARALLEL,),
        )(i_hbm, o_hbm)

    return k(table, idx2)


def kernel(atom_fea, nbr_fea, nbr_fea_idx,
           W_embed, b_embed, W_edge, b_edge,
           ln0_s, ln0_b, W1_0, b1_0, W2_0, b2_0,
           ln1_s, ln1_b, W1_1, b1_1, W2_1, b2_1,
           Wr1, br1, Wr2, br2, Wr3, br3):
    n, d_in = atom_fea.shape
    m = nbr_fea.shape[1]
    af = W_embed.shape[1]
    maf = m * af
    nsub = maf // 128
    n4 = n * nsub
    nblk = n // _BLK
    blk4 = _BLK * nsub

    nbr4 = nbr_fea.reshape(n, maf).astype(jnp.bfloat16).reshape(n4, 128)
    idx_flat = nbr_fea_idx.reshape(-1)

    eye = jnp.eye(af, dtype=jnp.float32)
    A8 = jnp.tile(eye, (1, 128 // af)).astype(jnp.bfloat16)

    def layer_consts(W1, b1, W2, b2):
        W1s, W1n, W1e = W1[:af], W1[af:2 * af], W1[2 * af:]
        C = W_edge @ W1e
        KC = jnp.kron(jnp.eye(128 // af, dtype=jnp.float32),
                      C).astype(jnp.bfloat16)
        W2S = jnp.tile(W2, (128 // af, 1)).astype(jnp.bfloat16)
        ba = (b1 + b_edge @ W1e).reshape(1, af)
        bx = (m * b2).reshape(1, af)
        return W1s, W1n, KC, W2S, ba, bx

    W1s0, W1n0, KC0, W2S0, ba0, bx0 = layer_consts(W1_0, b1_0, W2_0, b2_0)
    W1s1, W1n1, KC1, W2S1, ba1, bx1 = layer_consts(W1_1, b1_1, W2_1, b2_1)

    row = lambda shp: pl.BlockSpec(shp, lambda i: (i, 0))
    full = lambda shp: pl.BlockSpec(shp, lambda i: (0, 0))
    b16 = [jax.ShapeDtypeStruct((n, af), jnp.float32)] * 3

    x0, a0, g0 = pl.pallas_call(
        _embed_body,
        grid=(nblk,),
        in_specs=[row((_BLK, d_in)), full((d_in, af)), full((1, af)),
                  full((1, af)), full((1, af)), full((af, af)),
                  full((af, af)), full((1, af))],
        out_specs=[row((_BLK, af))] * 3,
        out_shape=b16,
    )(atom_fea, W_embed, b_embed.reshape(1, af),
      ln0_s.reshape(1, af), ln0_b.reshape(1, af), W1s0, W1n0, ba0)

    gath0 = _sc_gather(g0, idx_flat).reshape(n4, 128)

    x1, a1, g1 = pl.pallas_call(
        _layer_body,
        grid=(nblk,),
        in_specs=[row((blk4, 128)), row((blk4, 128)), row((_BLK, af)),
                  row((_BLK, af)), full((128, 128)), full((af, 128)),
                  full((128, af)), full((1, af)), full((1, af)),
                  full((1, af)), full((af, af)), full((af, af)),
                  full((1, af))],
        out_specs=[row((_BLK, af))] * 3,
        out_shape=b16,
    )(nbr4, gath0, a0, x0, KC0, A8, W2S0, bx0,
      ln1_s.reshape(1, af), ln1_b.reshape(1, af), W1s1, W1n1, ba1)

    gath1 = _sc_gather(g1, idx_flat).reshape(n4, 128)

    h = Wr1.shape[1]
    acc = pl.pallas_call(
        _final_body,
        grid=(nblk,),
        in_specs=[row((blk4, 128)), row((blk4, 128)), row((_BLK, af)),
                  row((_BLK, af)), full((128, 128)), full((af, 128)),
                  full((128, af)), full((1, af)),
                  full((af, h)), full((1, h)), full((h, h // 2)),
                  full((1, h // 2)), full((h // 2, 1)), full((1, 1))],
        out_specs=pl.BlockSpec((1, 1), lambda i: (0, 0)),
        out_shape=jax.ShapeDtypeStruct((1, 1), jnp.float32),
    )(nbr4, gath1, a1, x1, KC1, A8, W2S1, bx1,
      Wr1, br1.reshape(1, h), Wr2, br2.reshape(1, h // 2),
      Wr3, br3.reshape(1, 1))

    return acc[0, 0] / n


# speedup vs baseline: 1.4701x; 1.4696x over previous
import functools

import jax
import jax.numpy as jnp
from jax.experimental import pallas as pl
from jax.experimental.pallas import tpu as pltpu
from jax.experimental.pallas import tpu_sc as plsc

_BLK = 2000
_GW = 1600


def _f32dot(a, b):
    return jnp.dot(a, b, preferred_element_type=jnp.float32)


def _ln(x, s, b):
    mu = jnp.mean(x, axis=-1, keepdims=True)
    xc = x - mu
    var = jnp.mean(xc * xc, axis=-1, keepdims=True)
    return xc * jax.lax.rsqrt(var + 1e-6) * s + b


def _silu(x):
    return x * (0.5 + 0.5 * jnp.tanh(0.5 * x))


def _softplus(x):
    return jnp.maximum(x, 0.0) + jnp.log1p(jnp.exp(-jnp.abs(x)))


def _embed_body(atom_ref, We_ref, be_ref, lns_ref, lnb_ref, W1s_ref, W1n_ref,
                ba_ref, x_ref, a_ref, g_ref):
    x = _f32dot(atom_ref[...], We_ref[...]) + be_ref[...]
    x_ref[...] = x
    xn = _ln(x, lns_ref[...], lnb_ref[...])
    a_ref[...] = _f32dot(xn, W1s_ref[...]) + ba_ref[...]
    g_ref[...] = _f32dot(xn, W1n_ref[...])


def _edge_agg(nbr_ref, gath_ref, a_ref, KC_ref, A8_ref, W2S_ref):
    blk4 = nbr_ref.shape[0]
    blk = blk4 // 4
    aterm = _f32dot(a_ref[...].astype(jnp.bfloat16), A8_ref[...])
    aterm4 = jnp.broadcast_to(aterm[:, None, :], (blk, 4, 128))
    pre = (_f32dot(nbr_ref[...], KC_ref[...]) + gath_ref[...]
           + aterm4.reshape(blk4, 128))
    s = _silu(pre)
    ssum = jnp.sum(s.reshape(blk, 4, 128), axis=1)
    return _f32dot(ssum.astype(jnp.bfloat16), W2S_ref[...])


def _layer_body(nbr_ref, gath_ref, a_ref, x_ref, KC_ref, A8_ref, W2S_ref,
                bx_ref, lns_ref, lnb_ref, W1s_ref, W1n_ref, ba_ref,
                x1_ref, a1_ref, g1_ref):
    agg = _edge_agg(nbr_ref, gath_ref, a_ref, KC_ref, A8_ref, W2S_ref)
    x1 = x_ref[...] + agg + bx_ref[...]
    x1_ref[...] = x1
    xn = _ln(x1, lns_ref[...], lnb_ref[...])
    a1_ref[...] = _f32dot(xn, W1s_ref[...]) + ba_ref[...]
    g1_ref[...] = _f32dot(xn, W1n_ref[...])


def _final_body(nbr_ref, gath_ref, a_ref, x_ref, KC_ref, A8_ref, W2S_ref,
                bx_ref, Wr1_ref, br1_ref, Wr2_ref, br2_ref, Wr3_ref, br3_ref,
                acc_ref):
    agg = _edge_agg(nbr_ref, gath_ref, a_ref, KC_ref, A8_ref, W2S_ref)
    x2 = x_ref[...] + agg + bx_ref[...]
    h = _softplus(_f32dot(x2, Wr1_ref[...]) + br1_ref[...])
    t = _softplus(_f32dot(h, Wr2_ref[...]) + br2_ref[...])
    part = (jnp.sum(_f32dot(t, Wr3_ref[...]), keepdims=True)
            + t.shape[0] * br3_ref[...])
    i = pl.program_id(0)

    @pl.when(i == 0)
    def _():
        acc_ref[...] = part

    @pl.when(i > 0)
    def _():
        acc_ref[...] += part


def _sc_gather(table, idx_flat):
    num = idx_flat.shape[0]
    af = table.shape[1]
    mesh = plsc.VectorSubcoreMesh(core_axis_name="c", subcore_axis_name="s")
    idx2 = idx_flat.reshape(num // _GW, _GW)

    @functools.partial(
        pl.kernel,
        out_type=jax.ShapeDtypeStruct((num, af), table.dtype),
        mesh=mesh,
        compiler_params=pltpu.CompilerParams(use_tc_tiling_on_sc=False),
    )
    def k(table_hbm, i_hbm, o_hbm):
        def body(i_vmem, o_vmem):
            pltpu.sync_copy(table_hbm.at[i_vmem.at[0]], o_vmem)

        pltpu.emit_pipeline(
            body,
            grid=(num // _GW,),
            in_specs=[pl.BlockSpec((1, _GW), lambda i: (i, 0))],
            out_specs=[pl.BlockSpec((_GW, af), lambda i: (i, 0))],
            core_axis_name=("c", "s"),
            dimension_semantics=(pltpu.PARALLEL,),
        )(i_hbm, o_hbm)

    return k(table, idx2)


def kernel(atom_fea, nbr_fea, nbr_fea_idx,
           W_embed, b_embed, W_edge, b_edge,
           ln0_s, ln0_b, W1_0, b1_0, W2_0, b2_0,
           ln1_s, ln1_b, W1_1, b1_1, W2_1, b2_1,
           Wr1, br1, Wr2, br2, Wr3, br3):
    n, d_in = atom_fea.shape
    m = nbr_fea.shape[1]
    af = W_embed.shape[1]
    maf = m * af
    nsub = maf // 128
    n4 = n * nsub
    nblk = n // _BLK
    blk4 = _BLK * nsub

    nbr2d = jax.lax.optimization_barrier(
        nbr_fea.reshape(n, maf).astype(jnp.bfloat16))
    nbr4 = nbr2d.reshape(n4, 128)
    idx_flat = nbr_fea_idx.reshape(-1)

    eye = jnp.eye(af, dtype=jnp.float32)
    A8 = jnp.tile(eye, (1, 128 // af)).astype(jnp.bfloat16)

    def layer_consts(W1, b1, W2, b2):
        W1s, W1n, W1e = W1[:af], W1[af:2 * af], W1[2 * af:]
        C = W_edge @ W1e
        KC = jnp.kron(jnp.eye(128 // af, dtype=jnp.float32),
                      C).astype(jnp.bfloat16)
        W2S = jnp.tile(W2, (128 // af, 1)).astype(jnp.bfloat16)
        ba = (b1 + b_edge @ W1e).reshape(1, af)
        bx = (m * b2).reshape(1, af)
        return W1s, W1n, KC, W2S, ba, bx

    W1s0, W1n0, KC0, W2S0, ba0, bx0 = layer_consts(W1_0, b1_0, W2_0, b2_0)
    W1s1, W1n1, KC1, W2S1, ba1, bx1 = layer_consts(W1_1, b1_1, W2_1, b2_1)

    row = lambda shp: pl.BlockSpec(shp, lambda i: (i, 0))
    full = lambda shp: pl.BlockSpec(shp, lambda i: (0, 0))
    b16 = [jax.ShapeDtypeStruct((n, af), jnp.float32)] * 3

    x0, a0, g0 = pl.pallas_call(
        _embed_body,
        grid=(nblk,),
        in_specs=[row((_BLK, d_in)), full((d_in, af)), full((1, af)),
                  full((1, af)), full((1, af)), full((af, af)),
                  full((af, af)), full((1, af))],
        out_specs=[row((_BLK, af))] * 3,
        out_shape=b16,
    )(atom_fea, W_embed, b_embed.reshape(1, af),
      ln0_s.reshape(1, af), ln0_b.reshape(1, af), W1s0, W1n0, ba0)

    gath0 = _sc_gather(g0, idx_flat).reshape(n4, 128)

    x1, a1, g1 = pl.pallas_call(
        _layer_body,
        grid=(nblk,),
        in_specs=[row((blk4, 128)), row((blk4, 128)), row((_BLK, af)),
                  row((_BLK, af)), full((128, 128)), full((af, 128)),
                  full((128, af)), full((1, af)), full((1, af)),
                  full((1, af)), full((af, af)), full((af, af)),
                  full((1, af))],
        out_specs=[row((_BLK, af))] * 3,
        out_shape=b16,
    )(nbr4, gath0, a0, x0, KC0, A8, W2S0, bx0,
      ln1_s.reshape(1, af), ln1_b.reshape(1, af), W1s1, W1n1, ba1)

    gath1 = _sc_gather(g1, idx_flat).reshape(n4, 128)

    h = Wr1.shape[1]
    acc = pl.pallas_call(
        _final_body,
        grid=(nblk,),
        in_specs=[row((blk4, 128)), row((blk4, 128)), row((_BLK, af)),
                  row((_BLK, af)), full((128, 128)), full((af, 128)),
                  full((128, af)), full((1, af)),
                  full((af, h)), full((1, h)), full((h, h // 2)),
                  full((1, h // 2)), full((h // 2, 1)), full((1, 1))],
        out_specs=pl.BlockSpec((1, 1), lambda i: (0, 0)),
        out_shape=jax.ShapeDtypeStruct((1, 1), jnp.float32),
    )(nbr4, gath1, a1, x1, KC1, A8, W2S1, bx1,
      Wr1, br1.reshape(1, h), Wr2, br2.reshape(1, h // 2),
      Wr3, br3.reshape(1, 1))

    return acc[0, 0] / n
